# baseline (speedup 1.0000x reference)
import functools

import jax
import jax.numpy as jnp
from jax import lax
from jax.experimental import pallas as pl
from jax.experimental.pallas import tpu as pltpu
from jax.experimental.pallas import tpu_sc as plsc

TOPK = 2
NC, NS = 2, 16
NW = NC * NS
CH = 16


def _router_body(x_ref, gw_ref, s0_ref, s1_ref, sel_ref, rank_ref,
                 counts_ref, carry_ref, *, tb, e):
    i = pl.program_id(0)

    @pl.when(i == 0)
    def _():
        carry_ref[...] = jnp.zeros_like(carry_ref)

    x = x_ref[...]
    logits = lax.dot_general(x, gw_ref[...], (((1,), (1,)), ((), ())),
                             preferred_element_type=jnp.float32)
    m = jnp.max(logits, axis=1, keepdims=True)
    p = jnp.exp(logits - m)
    s = p / jnp.sum(p, axis=1, keepdims=True)

    lane = lax.broadcasted_iota(jnp.int32, (tb, e), 1)
    v0 = jnp.max(s, axis=1, keepdims=True)
    e0 = jnp.min(jnp.where(s == v0, lane, e), axis=1, keepdims=True)
    s_m = jnp.where(lane == e0, -jnp.inf, s)
    v1 = jnp.max(s_m, axis=1, keepdims=True)
    e1 = jnp.min(jnp.where(s_m == v1, lane, e), axis=1, keepdims=True)

    s0_ref[...] = x * v0
    s1_ref[...] = x * v1

    ecat = jnp.concatenate([e0, e1], axis=0)
    oh = (ecat == lax.broadcasted_iota(jnp.int32, (2 * tb, 128), 1))
    ohf = oh.astype(jnp.float32)
    row = lax.broadcasted_iota(jnp.int32, (2 * tb, 2 * tb), 0)
    col = lax.broadcasted_iota(jnp.int32, (2 * tb, 2 * tb), 1)
    tril = (col < row).astype(jnp.float32)
    excl = lax.dot_general(tril, ohf, (((1,), (0,)), ((), ())),
                           preferred_element_type=jnp.float32)
    carry = carry_ref[...]
    rank = jnp.sum((carry + excl) * ohf, axis=1, keepdims=True)
    sel_ref[...] = ecat
    rank_ref[...] = rank.astype(jnp.int32)
    new_carry = carry + jnp.sum(ohf, axis=0, keepdims=True)
    carry_ref[...] = new_carry
    counts_ref[...] = new_carry.astype(jnp.int32)


def _run_router(xf, gate_w, tb):
    n, d = xf.shape
    e = gate_w.shape[0]
    nblocks = n // tb
    body = functools.partial(_router_body, tb=tb, e=e)
    out_shape = [
        jax.ShapeDtypeStruct((n, d), jnp.float32),
        jax.ShapeDtypeStruct((n, d), jnp.float32),
        jax.ShapeDtypeStruct((2 * n, 1), jnp.int32),
        jax.ShapeDtypeStruct((2 * n, 1), jnp.int32),
        jax.ShapeDtypeStruct((1, 128), jnp.int32),
    ]
    return pl.pallas_call(
        body,
        grid=(nblocks,),
        in_specs=[
            pl.BlockSpec((tb, d), lambda i: (i, 0)),
            pl.BlockSpec((e, d), lambda i: (0, 0)),
        ],
        out_specs=[
            pl.BlockSpec((tb, d), lambda i: (i, 0)),
            pl.BlockSpec((tb, d), lambda i: (i, 0)),
            pl.BlockSpec((2 * tb, 1), lambda i: (i, 0)),
            pl.BlockSpec((2 * tb, 1), lambda i: (i, 0)),
            pl.BlockSpec((1, 128), lambda i: (0, 0)),
        ],
        out_shape=out_shape,
        scratch_shapes=[pltpu.VMEM((1, 128), jnp.float32)],
        compiler_params=pltpu.CompilerParams(
            dimension_semantics=("arbitrary",)),
    )(xf, gate_w)


def _ffn1_body(be_ref, x_ref, w1_ref, w3_ref, h_ref):
    x = x_ref[...]
    a = lax.dot_general(x, w1_ref[0], (((1,), (0,)), ((), ())),
                        preferred_element_type=jnp.float32)
    b = lax.dot_general(x, w3_ref[0], (((1,), (0,)), ((), ())),
                        preferred_element_type=jnp.float32)
    h_ref[...] = (a * jax.nn.sigmoid(a)) * b


def _run_ffn1_grouped(xr, w1, w3, be, rb, nh):
    p, d = xr.shape
    e, _, h = w1.shape
    nb = p // rb
    hb = h // nh
    spec = pltpu.PrefetchScalarGridSpec(
        num_scalar_prefetch=1,
        grid=(nh, nb),
        in_specs=[
            pl.BlockSpec((rb, d), lambda n, i, be: (i, 0)),
            pl.BlockSpec((1, d, hb), lambda n, i, be: (be[i], 0, n)),
            pl.BlockSpec((1, d, hb), lambda n, i, be: (be[i], 0, n)),
        ],
        out_specs=pl.BlockSpec((rb, hb), lambda n, i, be: (i, n)),
    )
    return pl.pallas_call(
        _ffn1_body,
        grid_spec=spec,
        out_shape=jax.ShapeDtypeStruct((p, h), jnp.float32),
        compiler_params=pltpu.CompilerParams(
            dimension_semantics=("arbitrary", "arbitrary")),
    )(be, xr, w1, w3)


def _ffn2_body(be_ref, h_ref, w2_ref, y_ref):
    y_ref[...] = lax.dot_general(h_ref[...], w2_ref[0],
                                 (((1,), (0,)), ((), ())),
                                 preferred_element_type=jnp.float32)


def _run_ffn2_grouped(hr, w2, be, rb):
    p, h = hr.shape
    e, _, d = w2.shape
    nb = p // rb
    spec = pltpu.PrefetchScalarGridSpec(
        num_scalar_prefetch=1,
        grid=(nb,),
        in_specs=[
            pl.BlockSpec((rb, h), lambda i, be: (i, 0)),
            pl.BlockSpec((1, h, d), lambda i, be: (be[i], 0, 0)),
        ],
        out_specs=pl.BlockSpec((rb, d), lambda i, be: (i, 0)),
    )
    return pl.pallas_call(
        _ffn2_body,
        grid_spec=spec,
        out_shape=jax.ShapeDtypeStruct((p, d), jnp.float32),
        compiler_params=pltpu.CompilerParams(
            dimension_semantics=("arbitrary",)),
    )(be, hr, w2)


def _ffns_body(x_ref, w1_ref, w3_ref, w2_ref, g0_ref, g1_ref, y_ref):
    x = x_ref[...]
    a = lax.dot_general(x, w1_ref[...], (((1,), (0,)), ((), ())),
                        preferred_element_type=jnp.float32)
    b = lax.dot_general(x, w3_ref[...], (((1,), (0,)), ((), ())),
                        preferred_element_type=jnp.float32)
    h = (a * jax.nn.sigmoid(a)) * b
    y = lax.dot_general(h, w2_ref[...], (((1,), (0,)), ((), ())),
                        preferred_element_type=jnp.float32)
    y_ref[...] = y + g0_ref[...] + g1_ref[...]


def _run_ffn_shared(xf, sw1, sw2, sw3, g0, g1, rb):
    n, d = xf.shape
    h = sw1.shape[1]
    nb = n // rb
    return pl.pallas_call(
        _ffns_body,
        grid=(nb,),
        in_specs=[
            pl.BlockSpec((rb, d), lambda i: (i, 0)),
            pl.BlockSpec((d, h), lambda i: (0, 0)),
            pl.BlockSpec((d, h), lambda i: (0, 0)),
            pl.BlockSpec((h, d), lambda i: (0, 0)),
            pl.BlockSpec((rb, d), lambda i: (i, 0)),
            pl.BlockSpec((rb, d), lambda i: (i, 0)),
        ],
        out_specs=pl.BlockSpec((rb, d), lambda i: (i, 0)),
        out_shape=jax.ShapeDtypeStruct((n, d), jnp.float32),
        compiler_params=pltpu.CompilerParams(
            dimension_semantics=("arbitrary",)),
    )(xf, sw1, sw3, sw2, g0, g1)


def _sc_dispatch(s0, s1, d0_3, d1_3, p):
    n, d = s0.shape
    nch = d0_3.shape[1]
    mesh = plsc.VectorSubcoreMesh(core_axis_name="c", subcore_axis_name="s")

    @functools.partial(
        pl.kernel,
        out_type=jax.ShapeDtypeStruct((p, d), jnp.float32),
        mesh=mesh,
        scratch_types=[
            pltpu.VMEM((nch, CH), jnp.int32),
            pltpu.VMEM((nch, CH), jnp.int32),
            pltpu.VMEM((CH, d), jnp.float32),
            pltpu.VMEM((CH, d), jnp.float32),
            pltpu.SemaphoreType.DMA,
            pltpu.SemaphoreType.DMA,
            pltpu.SemaphoreType.DMA,
            pltpu.SemaphoreType.DMA,
        ],
    )
    def disp(s0_hbm, s1_hbm, d0_hbm, d1_hbm, xr_hbm, idx0_v, idx1_v,
             bufa, bufb, sin_a, sin_b, sout_a, sout_b):
        wid = lax.axis_index("s") * NC + lax.axis_index("c")
        base = wid * (n // NW)
        pltpu.sync_copy(d0_hbm.at[wid], idx0_v)
        pltpu.sync_copy(d1_hbm.at[wid], idx1_v)
        srcs = [s0_hbm, s1_hbm]
        idxs = [idx0_v, idx1_v]
        bufs = [bufa, bufb]
        sins = [sin_a, sin_b]
        souts = [sout_a, sout_b]
        total = 2 * nch
        loads = [None] * total
        stores = [None] * total

        def start_load(t):
            k, c = t % 2, t // 2
            loads[t] = pltpu.async_copy(
                srcs[k].at[pl.ds(base + c * CH, CH)], bufs[t % 2],
                sins[t % 2])

        start_load(0)
        for t in range(total):
            if t + 1 < total:
                if t >= 1:
                    stores[t - 1].wait()
                start_load(t + 1)
            loads[t].wait()
            k, c = t % 2, t // 2
            stores[t] = pltpu.async_copy(
                bufs[t % 2], xr_hbm.at[idxs[k].at[c]], souts[t % 2])
        stores[total - 2].wait()
        stores[total - 1].wait()

    return disp(s0, s1, d0_3, d1_3)


def _sc_gather2(yr, d0_3, d1_3, n):
    pr, d = yr.shape
    nch = d0_3.shape[1]
    mesh = plsc.VectorSubcoreMesh(core_axis_name="c", subcore_axis_name="s")

    @functools.partial(
        pl.kernel,
        out_type=[jax.ShapeDtypeStruct((n, d), jnp.float32),
                  jax.ShapeDtypeStruct((n, d), jnp.float32)],
        mesh=mesh,
        scratch_types=[
            pltpu.VMEM((nch, CH), jnp.int32),
            pltpu.VMEM((nch, CH), jnp.int32),
            pltpu.VMEM((CH, d), jnp.float32),
            pltpu.VMEM((CH, d), jnp.float32),
            pltpu.SemaphoreType.DMA,
            pltpu.SemaphoreType.DMA,
            pltpu.SemaphoreType.DMA,
            pltpu.SemaphoreType.DMA,
        ],
    )
    def comb(yr_hbm, d0_hbm, d1_hbm, g0_hbm, g1_hbm,
             idx0_v, idx1_v, bufa, bufb, sin_a, sin_b, sout_a, sout_b):
        wid = lax.axis_index("s") * NC + lax.axis_index("c")
        base = wid * (n // NW)
        pltpu.sync_copy(d0_hbm.at[wid], idx0_v)
        pltpu.sync_copy(d1_hbm.at[wid], idx1_v)
        idxs = [idx0_v, idx1_v]
        outs = [g0_hbm, g1_hbm]
        bufs = [bufa, bufb]
        sins = [sin_a, sin_b]
        souts = [sout_a, sout_b]
        total = 2 * nch
        loads = [None] * total
        stores = [None] * total

        def start_load(t):
            k, c = t % 2, t // 2
            loads[t] = pltpu.async_copy(
                yr_hbm.at[idxs[k].at[c]], bufs[t % 2], sins[t % 2])

        start_load(0)
        for t in range(total):
            if t + 1 < total:
                if t >= 1:
                    stores[t - 1].wait()
                start_load(t + 1)
            loads[t].wait()
            k, c = t % 2, t // 2
            stores[t] = pltpu.async_copy(
                bufs[t % 2], outs[k].at[pl.ds(base + c * CH, CH)],
                souts[t % 2])
        stores[total - 2].wait()
        stores[total - 1].wait()

    return comb(yr, d0_3, d1_3)


def kernel(x, gate_w, w1, w2, w3, sw1, sw2, sw3):
    bs, slen, d = x.shape
    n = bs * slen
    e, _, h = w1.shape
    xf = x.reshape(n, d)

    tb = 256 if n % 256 == 0 else n
    rb = 128 if n % 256 == 0 else n
    nbr = (TOPK * n) // rb + e
    p = nbr * rb

    scaled0, scaled1, sel, rank, counts = _run_router(xf, gate_w, tb)

    counts8 = counts[0, :e]
    nblk = (counts8 + rb - 1) // rb
    cumb = jnp.cumsum(nblk)
    start = (cumb - nblk) * rb
    bi = jnp.arange(nbr, dtype=jnp.int32)
    be = jnp.sum((bi[:, None] >= cumb[None, :]).astype(jnp.int32), axis=1)
    be = jnp.minimum(be, e - 1).astype(jnp.int32)

    selr = sel.reshape(n // tb, 2, tb)
    rankr = rank.reshape(n // tb, 2, tb)
    sel0 = selr[:, 0, :].reshape(n)
    sel1 = selr[:, 1, :].reshape(n)
    rank0 = rankr[:, 0, :].reshape(n)
    rank1 = rankr[:, 1, :].reshape(n)
    dst0 = start[sel0] + rank0
    dst1 = start[sel1] + rank1

    nch = n // (NW * CH)
    d0_3 = dst0.reshape(NW, nch, CH)
    d1_3 = dst1.reshape(NW, nch, CH)

    xr = _sc_dispatch(scaled0, scaled1, d0_3, d1_3, p)
    hr = _run_ffn1_grouped(xr, w1, w3, be, rb, nh=2)
    yr = _run_ffn2_grouped(hr, w2, be, rb)
    g0, g1 = _sc_gather2(yr, d0_3, d1_3, n)
    out = _run_ffn_shared(xf, sw1[0], sw2[0], sw3[0], g0, g1, rb)
    return out.reshape(bs, slen, d)


# speedup vs baseline: 1.0530x; 1.0530x over previous
import functools

import jax
import jax.numpy as jnp
from jax import lax
from jax.experimental import pallas as pl
from jax.experimental.pallas import tpu as pltpu
from jax.experimental.pallas import tpu_sc as plsc

TOPK = 2
NC, NS = 2, 16
NW = NC * NS
CH = 16


def _router_body(x_ref, gw_ref, s0_ref, s1_ref, sel_ref, rank_ref,
                 counts_ref, carry_ref, *, tb, e):
    i = pl.program_id(0)

    @pl.when(i == 0)
    def _():
        carry_ref[...] = jnp.zeros_like(carry_ref)

    x = x_ref[...]
    logits = lax.dot_general(x, gw_ref[...], (((1,), (1,)), ((), ())),
                             preferred_element_type=jnp.float32)
    m = jnp.max(logits, axis=1, keepdims=True)
    p = jnp.exp(logits - m)
    s = p / jnp.sum(p, axis=1, keepdims=True)

    lane = lax.broadcasted_iota(jnp.int32, (tb, e), 1)
    v0 = jnp.max(s, axis=1, keepdims=True)
    e0 = jnp.min(jnp.where(s == v0, lane, e), axis=1, keepdims=True)
    s_m = jnp.where(lane == e0, -jnp.inf, s)
    v1 = jnp.max(s_m, axis=1, keepdims=True)
    e1 = jnp.min(jnp.where(s_m == v1, lane, e), axis=1, keepdims=True)

    s0_ref[...] = x * v0
    s1_ref[...] = x * v1

    ecat = jnp.concatenate([e0, e1], axis=0)
    oh = (ecat == lax.broadcasted_iota(jnp.int32, (2 * tb, 128), 1))
    ohf = oh.astype(jnp.float32)
    row = lax.broadcasted_iota(jnp.int32, (2 * tb, 2 * tb), 0)
    col = lax.broadcasted_iota(jnp.int32, (2 * tb, 2 * tb), 1)
    tril = (col < row).astype(jnp.float32)
    excl = lax.dot_general(tril, ohf, (((1,), (0,)), ((), ())),
                           preferred_element_type=jnp.float32)
    carry = carry_ref[...]
    rank = jnp.sum((carry + excl) * ohf, axis=1, keepdims=True)
    sel_ref[...] = ecat
    rank_ref[...] = rank.astype(jnp.int32)
    new_carry = carry + jnp.sum(ohf, axis=0, keepdims=True)
    carry_ref[...] = new_carry
    counts_ref[...] = new_carry.astype(jnp.int32)


def _run_router(xf, gate_w, tb):
    n, d = xf.shape
    e = gate_w.shape[0]
    nblocks = n // tb
    body = functools.partial(_router_body, tb=tb, e=e)
    out_shape = [
        jax.ShapeDtypeStruct((n, d), jnp.float32),
        jax.ShapeDtypeStruct((n, d), jnp.float32),
        jax.ShapeDtypeStruct((2 * n, 1), jnp.int32),
        jax.ShapeDtypeStruct((2 * n, 1), jnp.int32),
        jax.ShapeDtypeStruct((1, 128), jnp.int32),
    ]
    return pl.pallas_call(
        body,
        grid=(nblocks,),
        in_specs=[
            pl.BlockSpec((tb, d), lambda i: (i, 0)),
            pl.BlockSpec((e, d), lambda i: (0, 0)),
        ],
        out_specs=[
            pl.BlockSpec((tb, d), lambda i: (i, 0)),
            pl.BlockSpec((tb, d), lambda i: (i, 0)),
            pl.BlockSpec((2 * tb, 1), lambda i: (i, 0)),
            pl.BlockSpec((2 * tb, 1), lambda i: (i, 0)),
            pl.BlockSpec((1, 128), lambda i: (0, 0)),
        ],
        out_shape=out_shape,
        scratch_shapes=[pltpu.VMEM((1, 128), jnp.float32)],
        compiler_params=pltpu.CompilerParams(
            dimension_semantics=("arbitrary",)),
    )(xf, gate_w)


def _ffn1_body(be_ref, x_ref, w1_ref, w3_ref, h_ref):
    x = x_ref[...]
    a = lax.dot_general(x, w1_ref[0], (((1,), (0,)), ((), ())),
                        preferred_element_type=jnp.float32)
    b = lax.dot_general(x, w3_ref[0], (((1,), (0,)), ((), ())),
                        preferred_element_type=jnp.float32)
    h_ref[...] = (a * jax.nn.sigmoid(a)) * b


def _run_ffn1_grouped(xr, w1, w3, be, rb, nh):
    p, d = xr.shape
    e, _, h = w1.shape
    nb = p // rb
    hb = h // nh
    spec = pltpu.PrefetchScalarGridSpec(
        num_scalar_prefetch=1,
        grid=(nh, nb),
        in_specs=[
            pl.BlockSpec((rb, d), lambda n, i, be: (i, 0)),
            pl.BlockSpec((1, d, hb), lambda n, i, be: (be[i], 0, n)),
            pl.BlockSpec((1, d, hb), lambda n, i, be: (be[i], 0, n)),
        ],
        out_specs=pl.BlockSpec((rb, hb), lambda n, i, be: (i, n)),
    )
    return pl.pallas_call(
        _ffn1_body,
        grid_spec=spec,
        out_shape=jax.ShapeDtypeStruct((p, h), jnp.float32),
        compiler_params=pltpu.CompilerParams(
            dimension_semantics=("arbitrary", "arbitrary")),
    )(be, xr, w1, w3)


def _ffn2_body(be_ref, h_ref, w2_ref, y_ref):
    y_ref[...] = lax.dot_general(h_ref[...], w2_ref[0],
                                 (((1,), (0,)), ((), ())),
                                 preferred_element_type=jnp.float32)


def _run_ffn2_grouped(hr, w2, be, rb):
    p, h = hr.shape
    e, _, d = w2.shape
    nb = p // rb
    spec = pltpu.PrefetchScalarGridSpec(
        num_scalar_prefetch=1,
        grid=(nb,),
        in_specs=[
            pl.BlockSpec((rb, h), lambda i, be: (i, 0)),
            pl.BlockSpec((1, h, d), lambda i, be: (be[i], 0, 0)),
        ],
        out_specs=pl.BlockSpec((rb, d), lambda i, be: (i, 0)),
    )
    return pl.pallas_call(
        _ffn2_body,
        grid_spec=spec,
        out_shape=jax.ShapeDtypeStruct((p, d), jnp.float32),
        compiler_params=pltpu.CompilerParams(
            dimension_semantics=("arbitrary",)),
    )(be, hr, w2)


def _ffns_body(x_ref, w1_ref, w3_ref, w2_ref, y_ref):
    x = x_ref[...]
    a = lax.dot_general(x, w1_ref[...], (((1,), (0,)), ((), ())),
                        preferred_element_type=jnp.float32)
    b = lax.dot_general(x, w3_ref[...], (((1,), (0,)), ((), ())),
                        preferred_element_type=jnp.float32)
    h = (a * jax.nn.sigmoid(a)) * b
    y_ref[...] = lax.dot_general(h, w2_ref[...], (((1,), (0,)), ((), ())),
                                 preferred_element_type=jnp.float32)


def _run_ffn_shared(xf, sw1, sw2, sw3, rb):
    n, d = xf.shape
    h = sw1.shape[1]
    nb = n // rb
    return pl.pallas_call(
        _ffns_body,
        grid=(nb,),
        in_specs=[
            pl.BlockSpec((rb, d), lambda i: (i, 0)),
            pl.BlockSpec((d, h), lambda i: (0, 0)),
            pl.BlockSpec((d, h), lambda i: (0, 0)),
            pl.BlockSpec((h, d), lambda i: (0, 0)),
        ],
        out_specs=pl.BlockSpec((rb, d), lambda i: (i, 0)),
        out_shape=jax.ShapeDtypeStruct((n, d), jnp.float32),
        compiler_params=pltpu.CompilerParams(
            dimension_semantics=("arbitrary",)),
    )(xf, sw1, sw3, sw2)


def _sc_dispatch(s0, s1, d0_3, d1_3, p):
    n, d = s0.shape
    nch = d0_3.shape[1]
    mesh = plsc.VectorSubcoreMesh(core_axis_name="c", subcore_axis_name="s")

    @functools.partial(
        pl.kernel,
        out_type=jax.ShapeDtypeStruct((p, d), jnp.float32),
        mesh=mesh,
        scratch_types=[
            pltpu.VMEM((nch, CH), jnp.int32),
            pltpu.VMEM((nch, CH), jnp.int32),
            pltpu.VMEM((CH, d), jnp.float32),
            pltpu.VMEM((CH, d), jnp.float32),
            pltpu.SemaphoreType.DMA,
            pltpu.SemaphoreType.DMA,
            pltpu.SemaphoreType.DMA,
            pltpu.SemaphoreType.DMA,
        ],
    )
    def disp(s0_hbm, s1_hbm, d0_hbm, d1_hbm, xr_hbm, idx0_v, idx1_v,
             bufa, bufb, sin_a, sin_b, sout_a, sout_b):
        wid = lax.axis_index("s") * NC + lax.axis_index("c")
        base = wid * (n // NW)
        pltpu.sync_copy(d0_hbm.at[wid], idx0_v)
        pltpu.sync_copy(d1_hbm.at[wid], idx1_v)
        srcs = [s0_hbm, s1_hbm]
        idxs = [idx0_v, idx1_v]
        bufs = [bufa, bufb]
        sins = [sin_a, sin_b]
        souts = [sout_a, sout_b]
        total = 2 * nch
        loads = [None] * total
        stores = [None] * total

        def start_load(t):
            k, c = t % 2, t // 2
            loads[t] = pltpu.async_copy(
                srcs[k].at[pl.ds(base + c * CH, CH)], bufs[t % 2],
                sins[t % 2])

        start_load(0)
        for t in range(total):
            if t + 1 < total:
                if t >= 1:
                    stores[t - 1].wait()
                start_load(t + 1)
            loads[t].wait()
            k, c = t % 2, t // 2
            stores[t] = pltpu.async_copy(
                bufs[t % 2], xr_hbm.at[idxs[k].at[c]], souts[t % 2])
        stores[total - 2].wait()
        stores[total - 1].wait()

    return disp(s0, s1, d0_3, d1_3)


def _sc_combine(yr, ys, d0_3, d1_3):
    pr, d = yr.shape
    n = ys.shape[0]
    nch = d0_3.shape[1]
    ch = d0_3.shape[2]
    mesh = plsc.VectorSubcoreMesh(core_axis_name="c", subcore_axis_name="s")

    @functools.partial(
        pl.kernel,
        out_type=jax.ShapeDtypeStruct((n, d), jnp.float32),
        mesh=mesh,
        scratch_types=[
            pltpu.VMEM((nch, ch), jnp.int32),
            pltpu.VMEM((nch, ch), jnp.int32),
            pltpu.VMEM((ch, d), jnp.float32),
            pltpu.VMEM((ch, d), jnp.float32),
            pltpu.VMEM((ch, d), jnp.float32),
            pltpu.SemaphoreType.DMA,
            pltpu.SemaphoreType.DMA,
            pltpu.SemaphoreType.DMA,
        ],
    )
    def comb(yr_hbm, ys_hbm, d0_hbm, d1_hbm, out_hbm,
             idx0_v, idx1_v, b0, b1, bsh, sem0, sem1, sem2):
        wid = lax.axis_index("s") * NC + lax.axis_index("c")
        base = wid * (n // NW)
        pltpu.sync_copy(d0_hbm.at[wid], idx0_v)
        pltpu.sync_copy(d1_hbm.at[wid], idx1_v)
        nv = d // 16
        shift = nv.bit_length() - 1
        mask = nv - 1
        outcp = None
        for c in range(nch):
            cp0 = pltpu.async_copy(yr_hbm.at[idx0_v.at[c]], b0, sem0)
            cp1 = pltpu.async_copy(yr_hbm.at[idx1_v.at[c]], b1, sem1)
            if outcp is not None:
                outcp.wait()
            pltpu.sync_copy(ys_hbm.at[pl.ds(base + c * ch, ch)], bsh)
            cp0.wait()
            cp1.wait()

            @plsc.parallel_loop(0, ch * nv, unroll=8)
            def _(t):
                r = lax.shift_right_logical(t, shift)
                js = pl.multiple_of(
                    lax.shift_left(jnp.bitwise_and(t, mask), 4), 16)
                bsh[r, pl.ds(js, 16)] = (bsh[r, pl.ds(js, 16)]
                                         + b0[r, pl.ds(js, 16)]
                                         + b1[r, pl.ds(js, 16)])

            outcp = pltpu.async_copy(
                bsh, out_hbm.at[pl.ds(base + c * ch, ch)], sem2)
        outcp.wait()

    return comb(yr, ys, d0_3, d1_3)


def kernel(x, gate_w, w1, w2, w3, sw1, sw2, sw3):
    bs, slen, d = x.shape
    n = bs * slen
    e, _, h = w1.shape
    xf = x.reshape(n, d)

    tb = 256 if n % 256 == 0 else n
    rb = tb
    nbr = (TOPK * n) // rb + e
    p = nbr * rb

    scaled0, scaled1, sel, rank, counts = _run_router(xf, gate_w, tb)

    counts8 = counts[0, :e]
    nblk = (counts8 + rb - 1) // rb
    cumb = jnp.cumsum(nblk)
    start = (cumb - nblk) * rb
    bi = jnp.arange(nbr, dtype=jnp.int32)
    be = jnp.sum((bi[:, None] >= cumb[None, :]).astype(jnp.int32), axis=1)
    be = jnp.minimum(be, e - 1).astype(jnp.int32)

    selr = sel.reshape(n // tb, 2, tb)
    rankr = rank.reshape(n // tb, 2, tb)
    sel0 = selr[:, 0, :].reshape(n)
    sel1 = selr[:, 1, :].reshape(n)
    rank0 = rankr[:, 0, :].reshape(n)
    rank1 = rankr[:, 1, :].reshape(n)
    dst0 = start[sel0] + rank0
    dst1 = start[sel1] + rank1

    nch = n // (NW * CH)
    d0_3 = dst0.reshape(NW, nch, CH)
    d1_3 = dst1.reshape(NW, nch, CH)

    xr = _sc_dispatch(scaled0, scaled1, d0_3, d1_3, p)
    ys = _run_ffn_shared(xf, sw1[0], sw2[0], sw3[0], rb)
    hr = _run_ffn1_grouped(xr, w1, w3, be, rb, nh=2)
    yr = _run_ffn2_grouped(hr, w2, be, rb)
    out = _sc_combine(yr, ys, d0_3, d1_3)
    return out.reshape(bs, slen, d)
